# Initial kernel scaffold; baseline (speedup 1.0000x reference)
import jax
import jax.numpy as jnp
from jax.experimental import pallas as pl


def kernel(x, edge_index, edge_attr, W1_rel, b1_rel, W1_root, W2_rel, b2_rel, W2_root, Wfc, bfc):
    raise NotImplementedError("write your pallas kernel here")


# speedup vs baseline: 6.3177x; 6.3177x over previous
import functools

import jax
import jax.numpy as jnp
from jax import lax
from jax.experimental import pallas as pl
from jax.experimental.pallas import tpu as pltpu
from jax.experimental.pallas import tpu_sc as plsc

N = 10000
E = 320000
D = 128
C = 64

NC = 2
NS = 16
NW = NC * NS
EW = E // NW
K = 80
NCH = EW // K
NB = 25
NBLK = NCH // NB
ROWS0 = 624
ZR = 48
TAIL0 = NS * ROWS0
TAIL = N - TAIL0

_F32 = jnp.float32
_I32 = jnp.int32


def _sc_scatter_fn():
    mesh = plsc.VectorSubcoreMesh(
        core_axis_name="c", subcore_axis_name="s", num_cores=NC, num_subcores=NS
    )

    @functools.partial(
        pl.kernel,
        out_type=jax.ShapeDtypeStruct((NC, N, D), _F32),
        mesh=mesh,
        scratch_types=dict(
            src_v=pltpu.VMEM((NB, K), _I32),
            dst_v=pltpu.VMEM((NB, K), _I32),
            w_v=pltpu.VMEM((NB * K,), _F32),
            rows=pltpu.VMEM((K, D), _F32),
            zbuf=pltpu.VMEM((ZR, D), _F32),
            acc=pltpu.VMEM_SHARED((N, D), _F32),
        ),
    )
    def sc_scatter(x_hbm, src_hbm, dst_hbm, w_hbm, out_hbm,
                   src_v, dst_v, w_v, rows, zbuf, acc):
        c = lax.axis_index("c")
        s = lax.axis_index("s")
        wid = c * NS + s

        @pl.loop(0, ZR)
        def _(i):
            for j in range(D // 16):
                zbuf[i, pl.ds(j * 16, 16)] = jnp.zeros((16,), _F32)

        row0 = s * ROWS0
        for i in range(ROWS0 // ZR):
            pltpu.sync_copy(zbuf, acc.at[pl.ds(row0 + i * ZR, ZR)])

        @pl.when(s == NS - 1)
        def _():
            pltpu.sync_copy(zbuf.at[pl.ds(0, TAIL)], acc.at[pl.ds(TAIL0, TAIL)])

        plsc.subcore_barrier()

        @pl.loop(0, NBLK)
        def _(blk):
            pltpu.sync_copy(src_hbm.at[wid, blk], src_v)
            pltpu.sync_copy(dst_hbm.at[wid, blk], dst_v)
            pltpu.sync_copy(w_hbm.at[wid, blk], w_v)

            @pl.loop(0, NB)
            def _(g):
                pltpu.sync_copy(x_hbm.at[src_v.at[g]], rows)

                @pl.loop(0, K // 16)
                def _(t):
                    wvec = w_v[pl.ds(g * K + t * 16, 16)]
                    for l in range(16):
                        wb = jnp.full((16,), wvec[l], dtype=_F32)
                        row = t * 16 + l
                        for j in range(D // 16):
                            sl = pl.ds(j * 16, 16)
                            rows[row, sl] = rows[row, sl] * wb

                pltpu.sync_copy(rows, acc.at[dst_v.at[g]], add=True)

        plsc.subcore_barrier()

        for i in range(ROWS0 // ZR):
            r0 = row0 + i * ZR
            pltpu.sync_copy(acc.at[pl.ds(r0, ZR)], zbuf)
            pltpu.sync_copy(zbuf, out_hbm.at[c, pl.ds(r0, ZR)])

        @pl.when(s == NS - 1)
        def _():
            pltpu.sync_copy(acc.at[pl.ds(TAIL0, TAIL)], zbuf.at[pl.ds(0, TAIL)])
            pltpu.sync_copy(zbuf.at[pl.ds(0, TAIL)], out_hbm.at[c, pl.ds(TAIL0, TAIL)])

    return sc_scatter


_SC_SCATTER = _sc_scatter_fn()

BT = 2000


def _dotT(a, w):
    return lax.dot_general(a, w, (((1,), (1,)), ((), ())),
                           preferred_element_type=_F32)


def _layer_body(p_ref, x_ref, wrel_ref, b_ref, wroot_ref, o_ref):
    agg = p_ref[0] + p_ref[1]
    t = _dotT(agg, wrel_ref[...]) + _dotT(x_ref[...], wroot_ref[...]) + b_ref[...]
    o_ref[...] = jnp.maximum(t, 0.0)


def _tc_layer(p, x, w_rel, b_rel, w_root):
    return pl.pallas_call(
        _layer_body,
        grid=(N // BT,),
        in_specs=[
            pl.BlockSpec((NC, BT, D), lambda i: (0, i, 0)),
            pl.BlockSpec((BT, D), lambda i: (i, 0)),
            pl.BlockSpec((D, D), lambda i: (0, 0)),
            pl.BlockSpec((1, D), lambda i: (0, 0)),
            pl.BlockSpec((D, D), lambda i: (0, 0)),
        ],
        out_specs=pl.BlockSpec((BT, D), lambda i: (i, 0)),
        out_shape=jax.ShapeDtypeStruct((N, D), _F32),
    )(p, x, w_rel, b_rel.reshape(1, D), w_root)


def _final_body(p_ref, h_ref, wrel_ref, b_ref, wroot_ref, wfc_ref, bfc_ref, o_ref):
    agg = p_ref[0] + p_ref[1]
    h2 = jnp.maximum(
        _dotT(agg, wrel_ref[...]) + _dotT(h_ref[...], wroot_ref[...]) + b_ref[...],
        0.0,
    )
    o_ref[...] = _dotT(h2, wfc_ref[...]) + bfc_ref[...]


def _tc_final(p, h, w_rel, b_rel, w_root, wfc, bfc):
    return pl.pallas_call(
        _final_body,
        grid=(N // BT,),
        in_specs=[
            pl.BlockSpec((NC, BT, D), lambda i: (0, i, 0)),
            pl.BlockSpec((BT, D), lambda i: (i, 0)),
            pl.BlockSpec((D, D), lambda i: (0, 0)),
            pl.BlockSpec((1, D), lambda i: (0, 0)),
            pl.BlockSpec((D, D), lambda i: (0, 0)),
            pl.BlockSpec((C, D), lambda i: (0, 0)),
            pl.BlockSpec((1, C), lambda i: (0, 0)),
        ],
        out_specs=pl.BlockSpec((BT, C), lambda i: (i, 0)),
        out_shape=jax.ShapeDtypeStruct((N, C), _F32),
    )(p, h, w_rel, b_rel.reshape(1, D), w_root, wfc, bfc.reshape(1, C))


def kernel(x, edge_index, edge_attr, W1_rel, b1_rel, W1_root,
           W2_rel, b2_rel, W2_root, Wfc, bfc):
    src = edge_index[0].reshape(NW, NBLK, NB, K)
    dst = edge_index[1].reshape(NW, NBLK, NB, K)
    w = edge_attr.reshape(NW, NBLK, NB * K)

    p1 = _SC_SCATTER(x, src, dst, w)
    h1 = _tc_layer(p1, x, W1_rel, b1_rel, W1_root)
    p2 = _SC_SCATTER(h1, src, dst, w)
    return _tc_final(p2, h1, W2_rel, b2_rel, W2_root, Wfc, bfc)
